# baseline (speedup 1.0000x reference)
import jax
import jax.numpy as jnp
from jax import lax
from jax.experimental import pallas as pl
from jax.experimental.pallas import tpu as pltpu
from jax.experimental.pallas import tpu_sc as plsc

N = 10000
NP = 10112
M = 320000
MP = 327680
D = 128
RPAD = 512
REP = 16
REP2 = 2
ROWS = MP // D
NW = 32
RPW = ROWS // NW
NPT = NP // 16

_HI = jax.lax.Precision.HIGHEST


def _leaky(x):
    return jnp.maximum(x, 0.01 * x)


def _ln(x, g, b):
    mu = jnp.mean(x, axis=-1, keepdims=True)
    xc = x - mu
    var = jnp.mean(xc * xc, axis=-1, keepdims=True)
    return xc * jax.lax.rsqrt(var + 1e-5) * g + b


def _init_slice(zeros, agg_sh, s):
    pltpu.sync_copy(zeros.at[pl.ds(s * NPT, NPT)],
                    agg_sh.at[pl.ds(s * NPT, NPT)])


def _drain_slice(agg_sh, dst, s):
    @pl.when(s < 15)
    def _():
        pltpu.sync_copy(agg_sh.at[pl.ds(s * NPT, NPT)],
                        dst.at[pl.ds(s * NPT, NPT)])

    @pl.when(s == 15)
    def _():
        pltpu.sync_copy(agg_sh.at[pl.ds(15 * NPT, N - 15 * NPT)],
                        dst.at[pl.ds(15 * NPT, N - 15 * NPT)])


def _tca_body(feat, w_ent, b_ent, wf, wb, h0, gf, gb):
    h = _leaky(jnp.dot(feat[...], w_ent[...], precision=_HI) + b_ent[...])
    h0[...] = h
    gf[...] = jnp.dot(h, wf[...], precision=_HI)
    gb[...] = jnp.dot(h, wb[...], precision=_HI)


def _tc_a(feat, w_ent, b_ent, wf, wb):
    blk = 1000
    return pl.pallas_call(
        _tca_body,
        grid=(N // blk,),
        in_specs=[
            pl.BlockSpec((blk, 768), lambda i: (i, 0)),
            pl.BlockSpec((768, D), lambda i: (0, 0)),
            pl.BlockSpec((D,), lambda i: (0,)),
            pl.BlockSpec((D, D), lambda i: (0, 0)),
            pl.BlockSpec((D, D), lambda i: (0, 0)),
        ],
        out_specs=[pl.BlockSpec((blk, D), lambda i: (i, 0))] * 3,
        out_shape=[
            jax.ShapeDtypeStruct((N, D), jnp.float32),
            jax.ShapeDtypeStruct((NP, D), jnp.float32),
            jax.ShapeDtypeStruct((NP, D), jnp.float32),
        ],
    )(feat, w_ent, b_ent, wf, wb)


def _tct_body(rel, wf, bf, wb, bb, u2, bu, tf, tb, t2, t0):
    r = rel[...]
    tf[...] = jnp.dot(r, wf[...], precision=_HI) + bf[...]
    tb[...] = jnp.dot(r, wb[...], precision=_HI) + bb[...]
    t2[...] = jnp.dot(r, u2[...], precision=_HI) + bu[...]
    t0[...] = r


def _tc_t(rel512, wf, bf, wb, bb, u2, bu):
    return pl.pallas_call(
        _tct_body,
        grid=(REP,),
        in_specs=[
            pl.BlockSpec((RPAD, D), lambda i: (0, 0)),
            pl.BlockSpec((D, D), lambda i: (0, 0)),
            pl.BlockSpec((D,), lambda i: (0,)),
            pl.BlockSpec((D, D), lambda i: (0, 0)),
            pl.BlockSpec((D,), lambda i: (0,)),
            pl.BlockSpec((D, D), lambda i: (0, 0)),
            pl.BlockSpec((D,), lambda i: (0,)),
        ],
        out_specs=[
            pl.BlockSpec((RPAD, D), lambda i: (i, 0)),
            pl.BlockSpec((RPAD, D), lambda i: (i, 0)),
            pl.BlockSpec((RPAD, D), lambda i: (i % REP2, 0)),
            pl.BlockSpec((RPAD, D), lambda i: (i % REP2, 0)),
        ],
        out_shape=[
            jax.ShapeDtypeStruct((REP * RPAD, D), jnp.float32),
            jax.ShapeDtypeStruct((REP * RPAD, D), jnp.float32),
            jax.ShapeDtypeStruct((REP2 * RPAD, D), jnp.float32),
            jax.ShapeDtypeStruct((REP2 * RPAD, D), jnp.float32),
        ],
    )(rel512, wf, bf, wb, bb, u2, bu)


def _sc1_body(h2d, t2d, r2d, gf, gb, tfh, tbh, zeros,
              agg_out,
              hidx, tidx, ridx, buf, buf2, agg_sh, sem, sem2):
    c = lax.axis_index("c")
    s = lax.axis_index("s")
    wid = c * 16 + s

    _init_slice(zeros, agg_sh, s)
    base = wid * RPW
    plsc.subcore_barrier()

    def _outer(g, _):
        off = base + g * 8
        pltpu.sync_copy(h2d.at[pl.ds(off, 8)], hidx)
        pltpu.sync_copy(t2d.at[pl.ds(off, 8)], tidx)
        pltpu.sync_copy(r2d.at[pl.ds(off, 8)], ridx)
        lax.fori_loop(0, 8, _loop, 0)
        return 0

    def _loop(j, _):
        hv, tv, rv = hidx.at[j], tidx.at[j], ridx.at[j]
        c1 = pltpu.async_copy(gf.at[hv], buf, sem)
        c2 = pltpu.async_copy(tfh.at[rv], buf2, sem2)
        c1.wait()
        pltpu.sync_copy(buf, agg_sh.at[tv], add=True)
        c3 = pltpu.async_copy(gb.at[tv], buf, sem)
        c2.wait()
        pltpu.sync_copy(buf2, agg_sh.at[tv], add=True)
        c4 = pltpu.async_copy(tbh.at[rv], buf2, sem2)
        c3.wait()
        pltpu.sync_copy(buf, agg_sh.at[hv], add=True)
        c4.wait()
        pltpu.sync_copy(buf2, agg_sh.at[hv], add=True)
        return 0

    lax.fori_loop(0, RPW // 8, _outer, 0)

    plsc.subcore_barrier()

    _drain_slice(agg_sh, agg_out.at[c], s)


def _sc_1(h2d, t2d, r2d, gf, gb, tf, tb, zeros):
    mesh = plsc.VectorSubcoreMesh(core_axis_name="c", subcore_axis_name="s")
    return pl.kernel(
        _sc1_body,
        out_type=jax.ShapeDtypeStruct((2, N, D), jnp.float32),
        mesh=mesh,
        scratch_types=[
            pltpu.VMEM((8, D), jnp.int32),
            pltpu.VMEM((8, D), jnp.int32),
            pltpu.VMEM((8, D), jnp.int32),
            pltpu.VMEM((D, D), jnp.float32),
            pltpu.VMEM((D, D), jnp.float32),
            pltpu.VMEM_SHARED((NP, D), jnp.float32),
            pltpu.SemaphoreType.DMA,
            pltpu.SemaphoreType.DMA,
        ],
    )(h2d, t2d, r2d, gf, gb, tf, tb, zeros)


def _sc0_body(h2d, t2d, ones_h, zeros,
              cnt_out,
              hidx, tidx, ones_v, cnt_sh, sem):
    c = lax.axis_index("c")
    s = lax.axis_index("s")
    wid = c * 16 + s

    _init_slice(zeros, cnt_sh, s)
    pltpu.sync_copy(ones_h, ones_v)
    base = wid * RPW
    plsc.subcore_barrier()

    def _outer(g, _):
        off = base + g * 8
        pltpu.sync_copy(h2d.at[pl.ds(off, 8)], hidx)
        pltpu.sync_copy(t2d.at[pl.ds(off, 8)], tidx)
        lax.fori_loop(0, 8, _loop, 0)
        return 0

    def _loop(j, _):
        pltpu.sync_copy(ones_v, cnt_sh.at[tidx.at[j]], add=True)
        pltpu.sync_copy(ones_v, cnt_sh.at[hidx.at[j]], add=True)
        return 0

    lax.fori_loop(0, RPW // 8, _outer, 0)

    plsc.subcore_barrier()
    _drain_slice(cnt_sh, cnt_out.at[c], s)


def _sc_0(h2d, t2d, ones_h, zeros):
    mesh = plsc.VectorSubcoreMesh(core_axis_name="c", subcore_axis_name="s")
    return pl.kernel(
        _sc0_body,
        out_type=jax.ShapeDtypeStruct((2, N, D), jnp.float32),
        mesh=mesh,
        scratch_types=[
            pltpu.VMEM((8, D), jnp.int32),
            pltpu.VMEM((8, D), jnp.int32),
            pltpu.VMEM((D, D), jnp.float32),
            pltpu.VMEM_SHARED((NP, D), jnp.float32),
            pltpu.SemaphoreType.DMA,
        ],
    )(h2d, t2d, ones_h, zeros)


def _tcr_body(cnt, out):
    c = cnt[...]
    out[...] = (c[0, :, 0] + c[1, :, 0])[:, None]


def _tc_r(cnt):
    blk = 1000
    return pl.pallas_call(
        _tcr_body,
        grid=(N // blk,),
        in_specs=[pl.BlockSpec((2, blk, D), lambda i: (0, i, 0))],
        out_specs=pl.BlockSpec((blk, 1), lambda i: (i, 0)),
        out_shape=jax.ShapeDtypeStruct((N, 1), jnp.float32),
    )(cnt)


def _tcb_body(agg, cnt, h0, g, b, u1, u3, wf, wb,
              h1o, a1o, a3o, gfo, gbo):
    a = agg[0] + agg[1]
    m = a / (cnt[...] + 1e-7)
    h1 = _ln(_leaky(m) + h0[...], g[...], b[...])
    h1o[...] = h1
    a1o[...] = jnp.dot(h1, u1[...], precision=_HI)
    a3o[...] = jnp.dot(h1, u3[...], precision=_HI)
    gfo[...] = jnp.dot(h1, wf[...], precision=_HI)
    gbo[...] = jnp.dot(h1, wb[...], precision=_HI)


def _tc_b(agg1, cnt, h0, g, b, u1, u3, wf, wb):
    blk = 1000
    return pl.pallas_call(
        _tcb_body,
        grid=(N // blk,),
        in_specs=[
            pl.BlockSpec((2, blk, D), lambda i: (0, i, 0)),
            pl.BlockSpec((blk, 1), lambda i: (i, 0)),
            pl.BlockSpec((blk, D), lambda i: (i, 0)),
            pl.BlockSpec((D,), lambda i: (0,)),
            pl.BlockSpec((D,), lambda i: (0,)),
            pl.BlockSpec((D, D), lambda i: (0, 0)),
            pl.BlockSpec((D, D), lambda i: (0, 0)),
            pl.BlockSpec((D, D), lambda i: (0, 0)),
            pl.BlockSpec((D, D), lambda i: (0, 0)),
        ],
        out_specs=[pl.BlockSpec((blk, D), lambda i: (i, 0))] * 5,
        out_shape=[
            jax.ShapeDtypeStruct((N, D), jnp.float32),
            jax.ShapeDtypeStruct((NP, D), jnp.float32),
            jax.ShapeDtypeStruct((NP, D), jnp.float32),
            jax.ShapeDtypeStruct((NP, D), jnp.float32),
            jax.ShapeDtypeStruct((NP, D), jnp.float32),
        ],
    )(agg1, cnt, h0, g, b, u1, u3, wf, wb)


def _sc2_body(h2d, t2d, r2d, a1, a3, t2h, t0h,
              g1_out, g0_out,
              hidx, tidx, ridx, bufa, bufb, bufc, bufd, t2_sh, t0_sh,
              sem, sem2, sem3, sem4):
    c = lax.axis_index("c")
    s = lax.axis_index("s")
    wid = c * 16 + s
    base = wid * RPW

    @pl.when(s == 0)
    def _():
        pltpu.sync_copy(t2h, t2_sh)

    @pl.when(s == 1)
    def _():
        pltpu.sync_copy(t0h, t0_sh)

    plsc.subcore_barrier()

    def _outer(g, _):
        off = base + g * 8
        pltpu.sync_copy(h2d.at[pl.ds(off, 8)], hidx)
        pltpu.sync_copy(t2d.at[pl.ds(off, 8)], tidx)
        pltpu.sync_copy(r2d.at[pl.ds(off, 8)], ridx)
        lax.fori_loop(0, 8, lambda j, _: _loop(g * 8 + j, j), 0)
        return 0

    def _loop(row_j, j):
        row = base + row_j
        hv, tv, rv = hidx.at[j], tidx.at[j], ridx.at[j]
        c1 = pltpu.async_copy(a1.at[hv], bufa, sem)
        c2 = pltpu.async_copy(a3.at[tv], bufb, sem2)
        c3 = pltpu.async_copy(t2_sh.at[rv], bufc, sem3)
        c4 = pltpu.async_copy(t0_sh.at[rv], bufd, sem4)
        c1.wait()
        c2.wait()
        c3.wait()

        def _add(jj, _):
            for k in range(8):
                sl = pl.ds(k * 16, 16)
                bufa[jj, sl] = bufa[jj, sl] + bufb[jj, sl] + bufc[jj, sl]
            return 0

        lax.fori_loop(0, D, _add, 0)
        pltpu.sync_copy(bufa, g1_out.at[pl.ds(row * D, D)])
        c4.wait()
        pltpu.sync_copy(bufd, g0_out.at[pl.ds(row * D, D)])
        return 0

    lax.fori_loop(0, RPW // 8, _outer, 0)


def _sc_2(h2d, t2d, r2d, a1, a3, t2, t0):
    mesh = plsc.VectorSubcoreMesh(core_axis_name="c", subcore_axis_name="s")
    return pl.kernel(
        _sc2_body,
        out_type=[
            jax.ShapeDtypeStruct((MP, D), jnp.float32),
            jax.ShapeDtypeStruct((MP, D), jnp.float32),
        ],
        mesh=mesh,
        scratch_types=[
            pltpu.VMEM((8, D), jnp.int32),
            pltpu.VMEM((8, D), jnp.int32),
            pltpu.VMEM((8, D), jnp.int32),
            pltpu.VMEM((D, D), jnp.float32),
            pltpu.VMEM((D, D), jnp.float32),
            pltpu.VMEM((D, D), jnp.float32),
            pltpu.VMEM((D, D), jnp.float32),
            pltpu.VMEM_SHARED((REP2 * RPAD, D), jnp.float32),
            pltpu.VMEM_SHARED((REP2 * RPAD, D), jnp.float32),
            pltpu.SemaphoreType.DMA,
            pltpu.SemaphoreType.DMA,
            pltpu.SemaphoreType.DMA,
            pltpu.SemaphoreType.DMA,
        ],
    )(h2d, t2d, r2d, a1, a3, t2, t0)


def _tcc_body(g1, g0, g, b, wf, bf, wb, bb, pfo, pbo):
    x = _leaky(g1[...]) + g0[...]
    e1 = _ln(x, g[...], b[...])
    pfo[...] = jnp.dot(e1, wf[...], precision=_HI) + bf[...]
    pbo[...] = jnp.dot(e1, wb[...], precision=_HI) + bb[...]


def _tc_c(g1, g0, g, b, wf, bf, wb, bb):
    blk = 2048
    return pl.pallas_call(
        _tcc_body,
        grid=(MP // blk,),
        in_specs=[
            pl.BlockSpec((blk, D), lambda i: (i, 0)),
            pl.BlockSpec((blk, D), lambda i: (i, 0)),
            pl.BlockSpec((D,), lambda i: (0,)),
            pl.BlockSpec((D,), lambda i: (0,)),
            pl.BlockSpec((D, D), lambda i: (0, 0)),
            pl.BlockSpec((D,), lambda i: (0,)),
            pl.BlockSpec((D, D), lambda i: (0, 0)),
            pl.BlockSpec((D,), lambda i: (0,)),
        ],
        out_specs=[pl.BlockSpec((blk, D), lambda i: (i, 0))] * 2,
        out_shape=[jax.ShapeDtypeStruct((MP, D), jnp.float32)] * 2,
    )(g1, g0, g, b, wf, bf, wb, bb)


def _sc3_body(h2d, t2d, pf, pb, gf, gb, zeros,
              agg_out,
              hidx, tidx, buf, buf2, agg_sh, sem, sem2):
    c = lax.axis_index("c")
    s = lax.axis_index("s")
    wid = c * 16 + s

    _init_slice(zeros, agg_sh, s)
    base = wid * RPW
    plsc.subcore_barrier()

    def _outer(g, _):
        off = base + g * 8
        pltpu.sync_copy(h2d.at[pl.ds(off, 8)], hidx)
        pltpu.sync_copy(t2d.at[pl.ds(off, 8)], tidx)
        lax.fori_loop(0, 8, lambda j, _: _loop(g * 8 + j, j), 0)
        return 0

    def _loop(row_j, j):
        row = base + row_j
        hv, tv = hidx.at[j], tidx.at[j]
        c1 = pltpu.async_copy(pf.at[pl.ds(row * D, D)], buf, sem)
        c2 = pltpu.async_copy(gf.at[hv], buf2, sem2)
        c1.wait()
        pltpu.sync_copy(buf, agg_sh.at[tv], add=True)
        c3 = pltpu.async_copy(pb.at[pl.ds(row * D, D)], buf, sem)
        c2.wait()
        pltpu.sync_copy(buf2, agg_sh.at[tv], add=True)
        c4 = pltpu.async_copy(gb.at[tv], buf2, sem2)
        c3.wait()
        pltpu.sync_copy(buf, agg_sh.at[hv], add=True)
        c4.wait()
        pltpu.sync_copy(buf2, agg_sh.at[hv], add=True)
        return 0

    lax.fori_loop(0, RPW // 8, _outer, 0)

    plsc.subcore_barrier()
    _drain_slice(agg_sh, agg_out.at[c], s)


def _sc_3(h2d, t2d, pf, pb, gf, gb, zeros):
    mesh = plsc.VectorSubcoreMesh(core_axis_name="c", subcore_axis_name="s")
    return pl.kernel(
        _sc3_body,
        out_type=jax.ShapeDtypeStruct((2, N, D), jnp.float32),
        mesh=mesh,
        scratch_types=[
            pltpu.VMEM((8, D), jnp.int32),
            pltpu.VMEM((8, D), jnp.int32),
            pltpu.VMEM((D, D), jnp.float32),
            pltpu.VMEM((D, D), jnp.float32),
            pltpu.VMEM_SHARED((NP, D), jnp.float32),
            pltpu.SemaphoreType.DMA,
            pltpu.SemaphoreType.DMA,
        ],
    )(h2d, t2d, pf, pb, gf, gb, zeros)


def _tcd_body(agg, cnt, h1, g, b, h2o):
    a = agg[0] + agg[1]
    m = a / (cnt[...] + 1e-7)
    h2o[...] = _ln(_leaky(m) + h1[...], g[...], b[...])


def _tc_d(agg2, cnt, h1, g, b):
    blk = 1000
    return pl.pallas_call(
        _tcd_body,
        grid=(N // blk,),
        in_specs=[
            pl.BlockSpec((2, blk, D), lambda i: (0, i, 0)),
            pl.BlockSpec((blk, 1), lambda i: (i, 0)),
            pl.BlockSpec((blk, D), lambda i: (i, 0)),
            pl.BlockSpec((D,), lambda i: (0,)),
            pl.BlockSpec((D,), lambda i: (0,)),
        ],
        out_specs=pl.BlockSpec((blk, D), lambda i: (i, 0)),
        out_shape=jax.ShapeDtypeStruct((N, D), jnp.float32),
    )(agg2, cnt, h1, g, b)


def kernel(ht, r_tensor, entity_feat, params):
    p0 = params["layer0"]
    p1 = params["layer1"]

    pad = MP - M
    h2d = jnp.concatenate(
        [ht[:, 0], jnp.full((pad,), N, jnp.int32)]).reshape(ROWS, D)
    t2d = jnp.concatenate(
        [ht[:, 1], jnp.full((pad,), N, jnp.int32)]).reshape(ROWS, D)
    r_pad = jnp.concatenate(
        [r_tensor, jnp.full((pad,), 500, jnp.int32)])
    pos = jnp.arange(MP, dtype=jnp.int32)
    r2d = (r_pad + (pos % REP) * RPAD).reshape(ROWS, D)
    r2d2 = (r_pad + (pos % REP2) * RPAD).reshape(ROWS, D)
    rel512 = jnp.zeros((RPAD, D), jnp.float32).at[:500].set(params["rel_table"])
    zeros = jnp.zeros((NP, D), jnp.float32)
    ones_h = jnp.ones((D, D), jnp.float32)

    h0, gf1, gb1 = _tc_a(entity_feat, params["W_ent"], params["b_ent"],
                         p0["mpf_W"][:D], p0["mpb_W"][:D])
    tf1, tb1, t2, t0 = _tc_t(rel512, p0["mpf_W"][D:], p0["mpf_b"],
                             p0["mpb_W"][D:], p0["mpb_b"],
                             p0["eu_W"][D:2 * D], p0["eu_b"])
    agg1 = _sc_1(h2d, t2d, r2d, gf1, gb1, tf1, tb1, zeros)
    cnt2 = _sc_0(h2d, t2d, ones_h, zeros)
    cnt = _tc_r(cnt2)
    h1, a1, a3, gf2, gb2 = _tc_b(agg1, cnt, h0,
                                 p0["mp_ln_g"], p0["mp_ln_b"],
                                 p0["eu_W"][:D], p0["eu_W"][2 * D:],
                                 p1["mpf_W"][:D], p1["mpb_W"][:D])
    g1, g0 = _sc_2(h2d, t2d, r2d2, a1, a3, t2, t0)
    pf, pb = _tc_c(g1, g0, p0["eu_ln_g"], p0["eu_ln_b"],
                   p1["mpf_W"][D:], p1["mpf_b"],
                   p1["mpb_W"][D:], p1["mpb_b"])
    agg2 = _sc_3(h2d, t2d, pf, pb, gf2, gb2, zeros)
    h2 = _tc_d(agg2, cnt, h1, p1["mp_ln_g"], p1["mp_ln_b"])
    return h2


# speedup vs baseline: 1.0054x; 1.0054x over previous
import jax
import jax.numpy as jnp
from jax import lax
from jax.experimental import pallas as pl
from jax.experimental.pallas import tpu as pltpu
from jax.experimental.pallas import tpu_sc as plsc

N = 10000
NP = 10112
M = 320000
MP = 327680
D = 128
RPAD = 512
REP = 16
REP2 = 2
ROWS = MP // D
NW = 32
RPW = ROWS // NW
NPT = NP // 16

_HI = jax.lax.Precision.HIGHEST


def _leaky(x):
    return jnp.maximum(x, 0.01 * x)


def _ln(x, g, b):
    mu = jnp.mean(x, axis=-1, keepdims=True)
    xc = x - mu
    var = jnp.mean(xc * xc, axis=-1, keepdims=True)
    return xc * jax.lax.rsqrt(var + 1e-5) * g + b


def _init_slice(zeros, agg_sh, s):
    pltpu.sync_copy(zeros.at[pl.ds(s * NPT, NPT)],
                    agg_sh.at[pl.ds(s * NPT, NPT)])


def _drain_slice(agg_sh, dst, s):
    @pl.when(s < 15)
    def _():
        pltpu.sync_copy(agg_sh.at[pl.ds(s * NPT, NPT)],
                        dst.at[pl.ds(s * NPT, NPT)])

    @pl.when(s == 15)
    def _():
        pltpu.sync_copy(agg_sh.at[pl.ds(15 * NPT, N - 15 * NPT)],
                        dst.at[pl.ds(15 * NPT, N - 15 * NPT)])


def _tca_body(feat, w_ent, b_ent, wf, wb, h0, gf, gb):
    h = _leaky(jnp.dot(feat[...], w_ent[...], precision=_HI) + b_ent[...])
    h0[...] = h
    gf[...] = jnp.dot(h, wf[...], precision=_HI)
    gb[...] = jnp.dot(h, wb[...], precision=_HI)


def _tc_a(feat, w_ent, b_ent, wf, wb):
    blk = 1000
    return pl.pallas_call(
        _tca_body,
        grid=(N // blk,),
        in_specs=[
            pl.BlockSpec((blk, 768), lambda i: (i, 0)),
            pl.BlockSpec((768, D), lambda i: (0, 0)),
            pl.BlockSpec((D,), lambda i: (0,)),
            pl.BlockSpec((D, D), lambda i: (0, 0)),
            pl.BlockSpec((D, D), lambda i: (0, 0)),
        ],
        out_specs=[pl.BlockSpec((blk, D), lambda i: (i, 0))] * 3,
        out_shape=[
            jax.ShapeDtypeStruct((N, D), jnp.float32),
            jax.ShapeDtypeStruct((NP, D), jnp.float32),
            jax.ShapeDtypeStruct((NP, D), jnp.float32),
        ],
    )(feat, w_ent, b_ent, wf, wb)


def _tct_body(rel, wf, bf, wb, bb, u2, bu, tf, tb, t2, t0):
    r = rel[...]
    tf[...] = jnp.dot(r, wf[...], precision=_HI) + bf[...]
    tb[...] = jnp.dot(r, wb[...], precision=_HI) + bb[...]
    t2[...] = jnp.dot(r, u2[...], precision=_HI) + bu[...]
    t0[...] = r


def _tc_t(rel512, wf, bf, wb, bb, u2, bu):
    return pl.pallas_call(
        _tct_body,
        grid=(REP,),
        in_specs=[
            pl.BlockSpec((RPAD, D), lambda i: (0, 0)),
            pl.BlockSpec((D, D), lambda i: (0, 0)),
            pl.BlockSpec((D,), lambda i: (0,)),
            pl.BlockSpec((D, D), lambda i: (0, 0)),
            pl.BlockSpec((D,), lambda i: (0,)),
            pl.BlockSpec((D, D), lambda i: (0, 0)),
            pl.BlockSpec((D,), lambda i: (0,)),
        ],
        out_specs=[
            pl.BlockSpec((RPAD, D), lambda i: (0, 0)),
            pl.BlockSpec((RPAD, D), lambda i: (0, 0)),
            pl.BlockSpec((RPAD, D), lambda i: (i % REP2, 0)),
            pl.BlockSpec((RPAD, D), lambda i: (i % REP2, 0)),
        ],
        out_shape=[
            jax.ShapeDtypeStruct((RPAD, D), jnp.float32),
            jax.ShapeDtypeStruct((RPAD, D), jnp.float32),
            jax.ShapeDtypeStruct((REP2 * RPAD, D), jnp.float32),
            jax.ShapeDtypeStruct((REP2 * RPAD, D), jnp.float32),
        ],
    )(rel512, wf, bf, wb, bb, u2, bu)


def _sc1_body(h2d, t2d, r2d, gf, gb, tfh, tbh, zeros,
              agg_out,
              hidx, tidx, ridx, buf, buf2, agg_sh, tf_sh, tb_sh, sem, sem2):
    c = lax.axis_index("c")
    s = lax.axis_index("s")
    wid = c * 16 + s

    _init_slice(zeros, agg_sh, s)

    @pl.when(s == 0)
    def _():
        pltpu.sync_copy(tfh, tf_sh)

    @pl.when(s == 1)
    def _():
        pltpu.sync_copy(tbh, tb_sh)

    base = wid * RPW
    plsc.subcore_barrier()

    def _outer(g, _):
        off = base + g * 8
        pltpu.sync_copy(h2d.at[pl.ds(off, 8)], hidx)
        pltpu.sync_copy(t2d.at[pl.ds(off, 8)], tidx)
        pltpu.sync_copy(r2d.at[pl.ds(off, 8)], ridx)
        lax.fori_loop(0, 8, _loop, 0)
        return 0

    def _loop(j, _):
        hv, tv, rv = hidx.at[j], tidx.at[j], ridx.at[j]
        c1 = pltpu.async_copy(gf.at[hv], buf, sem)
        c2 = pltpu.async_copy(tf_sh.at[rv], buf2, sem2)
        c1.wait()
        pltpu.sync_copy(buf, agg_sh.at[tv], add=True)
        c3 = pltpu.async_copy(gb.at[tv], buf, sem)
        c2.wait()
        pltpu.sync_copy(buf2, agg_sh.at[tv], add=True)
        c4 = pltpu.async_copy(tb_sh.at[rv], buf2, sem2)
        c3.wait()
        pltpu.sync_copy(buf, agg_sh.at[hv], add=True)
        c4.wait()
        pltpu.sync_copy(buf2, agg_sh.at[hv], add=True)
        return 0

    lax.fori_loop(0, RPW // 8, _outer, 0)

    plsc.subcore_barrier()

    _drain_slice(agg_sh, agg_out.at[c], s)


def _sc_1(h2d, t2d, r2d, gf, gb, tf, tb, zeros):
    mesh = plsc.VectorSubcoreMesh(core_axis_name="c", subcore_axis_name="s")
    return pl.kernel(
        _sc1_body,
        out_type=jax.ShapeDtypeStruct((2, N, D), jnp.float32),
        mesh=mesh,
        scratch_types=[
            pltpu.VMEM((8, D), jnp.int32),
            pltpu.VMEM((8, D), jnp.int32),
            pltpu.VMEM((8, D), jnp.int32),
            pltpu.VMEM((D, D), jnp.float32),
            pltpu.VMEM((D, D), jnp.float32),
            pltpu.VMEM_SHARED((NP, D), jnp.float32),
            pltpu.VMEM_SHARED((RPAD, D), jnp.float32),
            pltpu.VMEM_SHARED((RPAD, D), jnp.float32),
            pltpu.SemaphoreType.DMA,
            pltpu.SemaphoreType.DMA,
        ],
    )(h2d, t2d, r2d, gf, gb, tf, tb, zeros)


def _sc0_body(h2d, t2d, ones_h, zeros,
              cnt_out,
              hidx, tidx, ones_v, cnt_sh, sem):
    c = lax.axis_index("c")
    s = lax.axis_index("s")
    wid = c * 16 + s

    _init_slice(zeros, cnt_sh, s)
    pltpu.sync_copy(ones_h, ones_v)
    base = wid * RPW
    plsc.subcore_barrier()

    def _outer(g, _):
        off = base + g * 8
        pltpu.sync_copy(h2d.at[pl.ds(off, 8)], hidx)
        pltpu.sync_copy(t2d.at[pl.ds(off, 8)], tidx)
        lax.fori_loop(0, 8, _loop, 0)
        return 0

    def _loop(j, _):
        pltpu.sync_copy(ones_v, cnt_sh.at[tidx.at[j]], add=True)
        pltpu.sync_copy(ones_v, cnt_sh.at[hidx.at[j]], add=True)
        return 0

    lax.fori_loop(0, RPW // 8, _outer, 0)

    plsc.subcore_barrier()
    _drain_slice(cnt_sh, cnt_out.at[c], s)


def _sc_0(h2d, t2d, ones_h, zeros):
    mesh = plsc.VectorSubcoreMesh(core_axis_name="c", subcore_axis_name="s")
    return pl.kernel(
        _sc0_body,
        out_type=jax.ShapeDtypeStruct((2, N, D), jnp.float32),
        mesh=mesh,
        scratch_types=[
            pltpu.VMEM((8, D), jnp.int32),
            pltpu.VMEM((8, D), jnp.int32),
            pltpu.VMEM((D, D), jnp.float32),
            pltpu.VMEM_SHARED((NP, D), jnp.float32),
            pltpu.SemaphoreType.DMA,
        ],
    )(h2d, t2d, ones_h, zeros)


def _tcr_body(cnt, out):
    c = cnt[...]
    out[...] = (c[0, :, 0] + c[1, :, 0])[:, None]


def _tc_r(cnt):
    blk = 1000
    return pl.pallas_call(
        _tcr_body,
        grid=(N // blk,),
        in_specs=[pl.BlockSpec((2, blk, D), lambda i: (0, i, 0))],
        out_specs=pl.BlockSpec((blk, 1), lambda i: (i, 0)),
        out_shape=jax.ShapeDtypeStruct((N, 1), jnp.float32),
    )(cnt)


def _tcb_body(agg, cnt, h0, g, b, u1, u3, wf, wb,
              h1o, a1o, a3o, gfo, gbo):
    a = agg[0] + agg[1]
    m = a / (cnt[...] + 1e-7)
    h1 = _ln(_leaky(m) + h0[...], g[...], b[...])
    h1o[...] = h1
    a1o[...] = jnp.dot(h1, u1[...], precision=_HI)
    a3o[...] = jnp.dot(h1, u3[...], precision=_HI)
    gfo[...] = jnp.dot(h1, wf[...], precision=_HI)
    gbo[...] = jnp.dot(h1, wb[...], precision=_HI)


def _tc_b(agg1, cnt, h0, g, b, u1, u3, wf, wb):
    blk = 1000
    return pl.pallas_call(
        _tcb_body,
        grid=(N // blk,),
        in_specs=[
            pl.BlockSpec((2, blk, D), lambda i: (0, i, 0)),
            pl.BlockSpec((blk, 1), lambda i: (i, 0)),
            pl.BlockSpec((blk, D), lambda i: (i, 0)),
            pl.BlockSpec((D,), lambda i: (0,)),
            pl.BlockSpec((D,), lambda i: (0,)),
            pl.BlockSpec((D, D), lambda i: (0, 0)),
            pl.BlockSpec((D, D), lambda i: (0, 0)),
            pl.BlockSpec((D, D), lambda i: (0, 0)),
            pl.BlockSpec((D, D), lambda i: (0, 0)),
        ],
        out_specs=[pl.BlockSpec((blk, D), lambda i: (i, 0))] * 5,
        out_shape=[
            jax.ShapeDtypeStruct((N, D), jnp.float32),
            jax.ShapeDtypeStruct((NP, D), jnp.float32),
            jax.ShapeDtypeStruct((NP, D), jnp.float32),
            jax.ShapeDtypeStruct((NP, D), jnp.float32),
            jax.ShapeDtypeStruct((NP, D), jnp.float32),
        ],
    )(agg1, cnt, h0, g, b, u1, u3, wf, wb)


def _sc2_body(h2d, t2d, r2d, a1, a3, t2h, t0h,
              g1_out, g0_out,
              hidx, tidx, ridx, bufa, bufb, bufc, bufd, t2_sh, t0_sh,
              sem, sem2, sem3, sem4):
    c = lax.axis_index("c")
    s = lax.axis_index("s")
    wid = c * 16 + s
    base = wid * RPW

    @pl.when(s == 0)
    def _():
        pltpu.sync_copy(t2h, t2_sh)

    @pl.when(s == 1)
    def _():
        pltpu.sync_copy(t0h, t0_sh)

    plsc.subcore_barrier()

    def _outer(g, _):
        off = base + g * 8
        pltpu.sync_copy(h2d.at[pl.ds(off, 8)], hidx)
        pltpu.sync_copy(t2d.at[pl.ds(off, 8)], tidx)
        pltpu.sync_copy(r2d.at[pl.ds(off, 8)], ridx)
        lax.fori_loop(0, 8, lambda j, _: _loop(g * 8 + j, j), 0)
        return 0

    def _loop(row_j, j):
        row = base + row_j
        hv, tv, rv = hidx.at[j], tidx.at[j], ridx.at[j]
        c1 = pltpu.async_copy(a1.at[hv], bufa, sem)
        c2 = pltpu.async_copy(a3.at[tv], bufb, sem2)
        c3 = pltpu.async_copy(t2_sh.at[rv], bufc, sem3)
        c4 = pltpu.async_copy(t0_sh.at[rv], bufd, sem4)
        c1.wait()
        c2.wait()
        c3.wait()

        def _add(jj, _):
            for k in range(8):
                sl = pl.ds(k * 16, 16)
                bufa[jj, sl] = bufa[jj, sl] + bufb[jj, sl] + bufc[jj, sl]
            return 0

        lax.fori_loop(0, D, _add, 0)
        pltpu.sync_copy(bufa, g1_out.at[pl.ds(row * D, D)])
        c4.wait()
        pltpu.sync_copy(bufd, g0_out.at[pl.ds(row * D, D)])
        return 0

    lax.fori_loop(0, RPW // 8, _outer, 0)


def _sc_2(h2d, t2d, r2d, a1, a3, t2, t0):
    mesh = plsc.VectorSubcoreMesh(core_axis_name="c", subcore_axis_name="s")
    return pl.kernel(
        _sc2_body,
        out_type=[
            jax.ShapeDtypeStruct((MP, D), jnp.float32),
            jax.ShapeDtypeStruct((MP, D), jnp.float32),
        ],
        mesh=mesh,
        scratch_types=[
            pltpu.VMEM((8, D), jnp.int32),
            pltpu.VMEM((8, D), jnp.int32),
            pltpu.VMEM((8, D), jnp.int32),
            pltpu.VMEM((D, D), jnp.float32),
            pltpu.VMEM((D, D), jnp.float32),
            pltpu.VMEM((D, D), jnp.float32),
            pltpu.VMEM((D, D), jnp.float32),
            pltpu.VMEM_SHARED((REP2 * RPAD, D), jnp.float32),
            pltpu.VMEM_SHARED((REP2 * RPAD, D), jnp.float32),
            pltpu.SemaphoreType.DMA,
            pltpu.SemaphoreType.DMA,
            pltpu.SemaphoreType.DMA,
            pltpu.SemaphoreType.DMA,
        ],
    )(h2d, t2d, r2d, a1, a3, t2, t0)


def _tcc_body(g1, g0, g, b, wf, bf, wb, bb, pfo, pbo):
    x = _leaky(g1[...]) + g0[...]
    e1 = _ln(x, g[...], b[...])
    pfo[...] = jnp.dot(e1, wf[...], precision=_HI) + bf[...]
    pbo[...] = jnp.dot(e1, wb[...], precision=_HI) + bb[...]


def _tc_c(g1, g0, g, b, wf, bf, wb, bb):
    blk = 2048
    return pl.pallas_call(
        _tcc_body,
        grid=(MP // blk,),
        in_specs=[
            pl.BlockSpec((blk, D), lambda i: (i, 0)),
            pl.BlockSpec((blk, D), lambda i: (i, 0)),
            pl.BlockSpec((D,), lambda i: (0,)),
            pl.BlockSpec((D,), lambda i: (0,)),
            pl.BlockSpec((D, D), lambda i: (0, 0)),
            pl.BlockSpec((D,), lambda i: (0,)),
            pl.BlockSpec((D, D), lambda i: (0, 0)),
            pl.BlockSpec((D,), lambda i: (0,)),
        ],
        out_specs=[pl.BlockSpec((blk, D), lambda i: (i, 0))] * 2,
        out_shape=[jax.ShapeDtypeStruct((MP, D), jnp.float32)] * 2,
    )(g1, g0, g, b, wf, bf, wb, bb)


def _sc3_body(h2d, t2d, pf, pb, gf, gb, zeros,
              agg_out,
              hidx, tidx, buf, buf2, agg_sh, sem, sem2):
    c = lax.axis_index("c")
    s = lax.axis_index("s")
    wid = c * 16 + s

    _init_slice(zeros, agg_sh, s)
    base = wid * RPW
    plsc.subcore_barrier()

    def _outer(g, _):
        off = base + g * 8
        pltpu.sync_copy(h2d.at[pl.ds(off, 8)], hidx)
        pltpu.sync_copy(t2d.at[pl.ds(off, 8)], tidx)
        lax.fori_loop(0, 8, lambda j, _: _loop(g * 8 + j, j), 0)
        return 0

    def _loop(row_j, j):
        row = base + row_j
        hv, tv = hidx.at[j], tidx.at[j]
        c1 = pltpu.async_copy(pf.at[pl.ds(row * D, D)], buf, sem)
        c2 = pltpu.async_copy(gf.at[hv], buf2, sem2)
        c1.wait()
        pltpu.sync_copy(buf, agg_sh.at[tv], add=True)
        c3 = pltpu.async_copy(pb.at[pl.ds(row * D, D)], buf, sem)
        c2.wait()
        pltpu.sync_copy(buf2, agg_sh.at[tv], add=True)
        c4 = pltpu.async_copy(gb.at[tv], buf2, sem2)
        c3.wait()
        pltpu.sync_copy(buf, agg_sh.at[hv], add=True)
        c4.wait()
        pltpu.sync_copy(buf2, agg_sh.at[hv], add=True)
        return 0

    lax.fori_loop(0, RPW // 8, _outer, 0)

    plsc.subcore_barrier()
    _drain_slice(agg_sh, agg_out.at[c], s)


def _sc_3(h2d, t2d, pf, pb, gf, gb, zeros):
    mesh = plsc.VectorSubcoreMesh(core_axis_name="c", subcore_axis_name="s")
    return pl.kernel(
        _sc3_body,
        out_type=jax.ShapeDtypeStruct((2, N, D), jnp.float32),
        mesh=mesh,
        scratch_types=[
            pltpu.VMEM((8, D), jnp.int32),
            pltpu.VMEM((8, D), jnp.int32),
            pltpu.VMEM((D, D), jnp.float32),
            pltpu.VMEM((D, D), jnp.float32),
            pltpu.VMEM_SHARED((NP, D), jnp.float32),
            pltpu.SemaphoreType.DMA,
            pltpu.SemaphoreType.DMA,
        ],
    )(h2d, t2d, pf, pb, gf, gb, zeros)


def _tcd_body(agg, cnt, h1, g, b, h2o):
    a = agg[0] + agg[1]
    m = a / (cnt[...] + 1e-7)
    h2o[...] = _ln(_leaky(m) + h1[...], g[...], b[...])


def _tc_d(agg2, cnt, h1, g, b):
    blk = 1000
    return pl.pallas_call(
        _tcd_body,
        grid=(N // blk,),
        in_specs=[
            pl.BlockSpec((2, blk, D), lambda i: (0, i, 0)),
            pl.BlockSpec((blk, 1), lambda i: (i, 0)),
            pl.BlockSpec((blk, D), lambda i: (i, 0)),
            pl.BlockSpec((D,), lambda i: (0,)),
            pl.BlockSpec((D,), lambda i: (0,)),
        ],
        out_specs=pl.BlockSpec((blk, D), lambda i: (i, 0)),
        out_shape=jax.ShapeDtypeStruct((N, D), jnp.float32),
    )(agg2, cnt, h1, g, b)


def kernel(ht, r_tensor, entity_feat, params):
    p0 = params["layer0"]
    p1 = params["layer1"]

    pad = MP - M
    h2d = jnp.concatenate(
        [ht[:, 0], jnp.full((pad,), N, jnp.int32)]).reshape(ROWS, D)
    t2d = jnp.concatenate(
        [ht[:, 1], jnp.full((pad,), N, jnp.int32)]).reshape(ROWS, D)
    r_pad = jnp.concatenate(
        [r_tensor, jnp.full((pad,), 500, jnp.int32)])
    pos = jnp.arange(MP, dtype=jnp.int32)
    r2d = r_pad.reshape(ROWS, D)
    r2d2 = (r_pad + (pos % REP2) * RPAD).reshape(ROWS, D)
    rel512 = jnp.zeros((RPAD, D), jnp.float32).at[:500].set(params["rel_table"])
    zeros = jnp.zeros((NP, D), jnp.float32)
    ones_h = jnp.ones((D, D), jnp.float32)

    h0, gf1, gb1 = _tc_a(entity_feat, params["W_ent"], params["b_ent"],
                         p0["mpf_W"][:D], p0["mpb_W"][:D])
    tf1, tb1, t2, t0 = _tc_t(rel512, p0["mpf_W"][D:], p0["mpf_b"],
                             p0["mpb_W"][D:], p0["mpb_b"],
                             p0["eu_W"][D:2 * D], p0["eu_b"])
    agg1 = _sc_1(h2d, t2d, r2d, gf1, gb1, tf1, tb1, zeros)
    cnt2 = _sc_0(h2d, t2d, ones_h, zeros)
    cnt = _tc_r(cnt2)
    h1, a1, a3, gf2, gb2 = _tc_b(agg1, cnt, h0,
                                 p0["mp_ln_g"], p0["mp_ln_b"],
                                 p0["eu_W"][:D], p0["eu_W"][2 * D:],
                                 p1["mpf_W"][:D], p1["mpb_W"][:D])
    g1, g0 = _sc_2(h2d, t2d, r2d2, a1, a3, t2, t0)
    pf, pb = _tc_c(g1, g0, p0["eu_ln_g"], p0["eu_ln_b"],
                   p1["mpf_W"][D:], p1["mpf_b"],
                   p1["mpb_W"][D:], p1["mpb_b"])
    agg2 = _sc_3(h2d, t2d, pf, pb, gf2, gb2, zeros)
    h2 = _tc_d(agg2, cnt, h1, p1["mp_ln_g"], p1["mp_ln_b"])
    return h2


# speedup vs baseline: 1.0075x; 1.0020x over previous
import jax
import jax.numpy as jnp
from jax import lax
from jax.experimental import pallas as pl
from jax.experimental.pallas import tpu as pltpu
from jax.experimental.pallas import tpu_sc as plsc

N = 10000
NP = 10112
M = 320000
MP = 327680
D = 128
RPAD = 512
REP = 16
REP2 = 2
ROWS = MP // D
NW = 32
RPW = ROWS // NW
NPT = NP // 16

_HI = jax.lax.Precision.HIGHEST


def _leaky(x):
    return jnp.maximum(x, 0.01 * x)


def _ln(x, g, b):
    mu = jnp.mean(x, axis=-1, keepdims=True)
    xc = x - mu
    var = jnp.mean(xc * xc, axis=-1, keepdims=True)
    return xc * jax.lax.rsqrt(var + 1e-5) * g + b


def _init_slice(zeros, agg_sh, s):
    pltpu.sync_copy(zeros.at[pl.ds(s * NPT, NPT)],
                    agg_sh.at[pl.ds(s * NPT, NPT)])


def _drain_slice(agg_sh, dst, s):
    @pl.when(s < 15)
    def _():
        pltpu.sync_copy(agg_sh.at[pl.ds(s * NPT, NPT)],
                        dst.at[pl.ds(s * NPT, NPT)])

    @pl.when(s == 15)
    def _():
        pltpu.sync_copy(agg_sh.at[pl.ds(15 * NPT, N - 15 * NPT)],
                        dst.at[pl.ds(15 * NPT, N - 15 * NPT)])


def _tca_body(feat, w_ent, b_ent, wf, wb, h0, gf, gb):
    h = _leaky(jnp.dot(feat[...], w_ent[...], precision=_HI) + b_ent[...])
    h0[...] = h
    gf[...] = jnp.dot(h, wf[...], precision=_HI)
    gb[...] = jnp.dot(h, wb[...], precision=_HI)


def _tc_a(feat, w_ent, b_ent, wf, wb):
    blk = 1000
    return pl.pallas_call(
        _tca_body,
        grid=(N // blk,),
        in_specs=[
            pl.BlockSpec((blk, 768), lambda i: (i, 0)),
            pl.BlockSpec((768, D), lambda i: (0, 0)),
            pl.BlockSpec((D,), lambda i: (0,)),
            pl.BlockSpec((D, D), lambda i: (0, 0)),
            pl.BlockSpec((D, D), lambda i: (0, 0)),
        ],
        out_specs=[pl.BlockSpec((blk, D), lambda i: (i, 0))] * 3,
        out_shape=[
            jax.ShapeDtypeStruct((N, D), jnp.float32),
            jax.ShapeDtypeStruct((NP, D), jnp.float32),
            jax.ShapeDtypeStruct((NP, D), jnp.float32),
        ],
    )(feat, w_ent, b_ent, wf, wb)


def _tct_body(rel, wf, bf, wb, bb, u2, bu, tf, tb, t2, t0):
    r = rel[...]
    tf[...] = jnp.dot(r, wf[...], precision=_HI) + bf[...]
    tb[...] = jnp.dot(r, wb[...], precision=_HI) + bb[...]
    t2[...] = jnp.dot(r, u2[...], precision=_HI) + bu[...]
    t0[...] = r


def _tc_t(rel512, wf, bf, wb, bb, u2, bu):
    return pl.pallas_call(
        _tct_body,
        grid=(REP,),
        in_specs=[
            pl.BlockSpec((RPAD, D), lambda i: (0, 0)),
            pl.BlockSpec((D, D), lambda i: (0, 0)),
            pl.BlockSpec((D,), lambda i: (0,)),
            pl.BlockSpec((D, D), lambda i: (0, 0)),
            pl.BlockSpec((D,), lambda i: (0,)),
            pl.BlockSpec((D, D), lambda i: (0, 0)),
            pl.BlockSpec((D,), lambda i: (0,)),
        ],
        out_specs=[
            pl.BlockSpec((RPAD, D), lambda i: (0, 0)),
            pl.BlockSpec((RPAD, D), lambda i: (0, 0)),
            pl.BlockSpec((RPAD, D), lambda i: (i % REP2, 0)),
            pl.BlockSpec((RPAD, D), lambda i: (i % REP2, 0)),
        ],
        out_shape=[
            jax.ShapeDtypeStruct((RPAD, D), jnp.float32),
            jax.ShapeDtypeStruct((RPAD, D), jnp.float32),
            jax.ShapeDtypeStruct((REP2 * RPAD, D), jnp.float32),
            jax.ShapeDtypeStruct((REP2 * RPAD, D), jnp.float32),
        ],
    )(rel512, wf, bf, wb, bb, u2, bu)


def _sc1_body(h2d, t2d, r2d, gf, gb, tfh, tbh, zeros,
              agg_out,
              hidx, tidx, ridx, buf, buf2, agg_sh, tf_sh, tb_sh, sem, sem2):
    c = lax.axis_index("c")
    s = lax.axis_index("s")
    wid = c * 16 + s

    _init_slice(zeros, agg_sh, s)

    @pl.when(s == 0)
    def _():
        pltpu.sync_copy(tfh, tf_sh)

    @pl.when(s == 1)
    def _():
        pltpu.sync_copy(tbh, tb_sh)

    base = wid * RPW
    plsc.subcore_barrier()

    def _outer(g, _):
        off = base + g * 8
        pltpu.sync_copy(h2d.at[pl.ds(off, 8)], hidx)
        pltpu.sync_copy(t2d.at[pl.ds(off, 8)], tidx)
        pltpu.sync_copy(r2d.at[pl.ds(off, 8)], ridx)
        lax.fori_loop(0, 8, _loop, 0)
        return 0

    def _loop(j, _):
        hv, tv, rv = hidx.at[j], tidx.at[j], ridx.at[j]
        c1 = pltpu.async_copy(gf.at[hv], buf, sem)
        c2 = pltpu.async_copy(tf_sh.at[rv], buf2, sem2)
        c1.wait()
        pltpu.sync_copy(buf, agg_sh.at[tv], add=True)
        c3 = pltpu.async_copy(gb.at[tv], buf, sem)
        c2.wait()
        pltpu.sync_copy(buf2, agg_sh.at[tv], add=True)
        c4 = pltpu.async_copy(tb_sh.at[rv], buf2, sem2)
        c3.wait()
        pltpu.sync_copy(buf, agg_sh.at[hv], add=True)
        c4.wait()
        pltpu.sync_copy(buf2, agg_sh.at[hv], add=True)
        return 0

    lax.fori_loop(0, RPW // 8, _outer, 0)

    plsc.subcore_barrier()

    _drain_slice(agg_sh, agg_out.at[c], s)


def _sc_1(h2d, t2d, r2d, gf, gb, tf, tb, zeros):
    mesh = plsc.VectorSubcoreMesh(core_axis_name="c", subcore_axis_name="s")
    return pl.kernel(
        _sc1_body,
        out_type=jax.ShapeDtypeStruct((2, N, D), jnp.float32),
        mesh=mesh,
        scratch_types=[
            pltpu.VMEM((8, D), jnp.int32),
            pltpu.VMEM((8, D), jnp.int32),
            pltpu.VMEM((8, D), jnp.int32),
            pltpu.VMEM((D, D), jnp.float32),
            pltpu.VMEM((D, D), jnp.float32),
            pltpu.VMEM_SHARED((NP, D), jnp.float32),
            pltpu.VMEM_SHARED((RPAD, D), jnp.float32),
            pltpu.VMEM_SHARED((RPAD, D), jnp.float32),
            pltpu.SemaphoreType.DMA,
            pltpu.SemaphoreType.DMA,
        ],
    )(h2d, t2d, r2d, gf, gb, tf, tb, zeros)


def _sc0_body(h2d, t2d, ones_h, zeros,
              cnt_out,
              hidx, tidx, ones_v, cnt_sh, sem):
    c = lax.axis_index("c")
    s = lax.axis_index("s")
    wid = c * 16 + s

    _init_slice(zeros, cnt_sh, s)
    pltpu.sync_copy(ones_h, ones_v)
    base = wid * RPW
    plsc.subcore_barrier()

    def _outer(g, _):
        off = base + g * 8
        pltpu.sync_copy(h2d.at[pl.ds(off, 8)], hidx)
        pltpu.sync_copy(t2d.at[pl.ds(off, 8)], tidx)
        lax.fori_loop(0, 8, _loop, 0)
        return 0

    def _loop(j, _):
        pltpu.sync_copy(ones_v, cnt_sh.at[tidx.at[j]], add=True)
        pltpu.sync_copy(ones_v, cnt_sh.at[hidx.at[j]], add=True)
        return 0

    lax.fori_loop(0, RPW // 8, _outer, 0)

    plsc.subcore_barrier()
    _drain_slice(cnt_sh, cnt_out.at[c], s)


def _sc_0(h2d, t2d, ones_h, zeros):
    mesh = plsc.VectorSubcoreMesh(core_axis_name="c", subcore_axis_name="s")
    return pl.kernel(
        _sc0_body,
        out_type=jax.ShapeDtypeStruct((2, N, D), jnp.float32),
        mesh=mesh,
        scratch_types=[
            pltpu.VMEM((8, D), jnp.int32),
            pltpu.VMEM((8, D), jnp.int32),
            pltpu.VMEM((D, D), jnp.float32),
            pltpu.VMEM_SHARED((NP, D), jnp.float32),
            pltpu.SemaphoreType.DMA,
        ],
    )(h2d, t2d, ones_h, zeros)


def _tcr_body(cnt, out):
    c = cnt[...]
    out[...] = (c[0, :, 0] + c[1, :, 0])[:, None]


def _tc_r(cnt):
    blk = 1000
    return pl.pallas_call(
        _tcr_body,
        grid=(N // blk,),
        in_specs=[pl.BlockSpec((2, blk, D), lambda i: (0, i, 0))],
        out_specs=pl.BlockSpec((blk, 1), lambda i: (i, 0)),
        out_shape=jax.ShapeDtypeStruct((N, 1), jnp.float32),
    )(cnt)


def _tcb_body(agg, cnt, h0, g, b, u1, u3, wf, wb,
              h1o, a1o, a3o, gfo, gbo):
    a = agg[0] + agg[1]
    m = a / (cnt[...] + 1e-7)
    h1 = _ln(_leaky(m) + h0[...], g[...], b[...])
    h1o[...] = h1
    a1o[...] = jnp.dot(h1, u1[...], precision=_HI)
    a3o[...] = jnp.dot(h1, u3[...], precision=_HI)
    gfo[...] = jnp.dot(h1, wf[...], precision=_HI)
    gbo[...] = jnp.dot(h1, wb[...], precision=_HI)


def _tc_b(agg1, cnt, h0, g, b, u1, u3, wf, wb):
    blk = 1000
    return pl.pallas_call(
        _tcb_body,
        grid=(N // blk,),
        in_specs=[
            pl.BlockSpec((2, blk, D), lambda i: (0, i, 0)),
            pl.BlockSpec((blk, 1), lambda i: (i, 0)),
            pl.BlockSpec((blk, D), lambda i: (i, 0)),
            pl.BlockSpec((D,), lambda i: (0,)),
            pl.BlockSpec((D,), lambda i: (0,)),
            pl.BlockSpec((D, D), lambda i: (0, 0)),
            pl.BlockSpec((D, D), lambda i: (0, 0)),
            pl.BlockSpec((D, D), lambda i: (0, 0)),
            pl.BlockSpec((D, D), lambda i: (0, 0)),
        ],
        out_specs=[pl.BlockSpec((blk, D), lambda i: (i, 0))] * 5,
        out_shape=[
            jax.ShapeDtypeStruct((N, D), jnp.float32),
            jax.ShapeDtypeStruct((NP, D), jnp.float32),
            jax.ShapeDtypeStruct((NP, D), jnp.float32),
            jax.ShapeDtypeStruct((NP, D), jnp.float32),
            jax.ShapeDtypeStruct((NP, D), jnp.float32),
        ],
    )(agg1, cnt, h0, g, b, u1, u3, wf, wb)


def _sc2_body(h2d, t2d, r2d, a1, a3, t2h, t0h,
              g1_out, g0_out,
              hidx, tidx, ridx, bufa, bufb, bufc, bufd, t2_sh, t0_sh,
              sem, sem2, sem3, sem4):
    c = lax.axis_index("c")
    s = lax.axis_index("s")
    wid = c * 16 + s
    base = wid * RPW

    @pl.when(s == 0)
    def _():
        pltpu.sync_copy(t2h, t2_sh)

    @pl.when(s == 1)
    def _():
        pltpu.sync_copy(t0h, t0_sh)

    plsc.subcore_barrier()

    def _outer(g, _):
        off = base + g * 8
        pltpu.sync_copy(h2d.at[pl.ds(off, 8)], hidx)
        pltpu.sync_copy(t2d.at[pl.ds(off, 8)], tidx)
        pltpu.sync_copy(r2d.at[pl.ds(off, 8)], ridx)
        lax.fori_loop(0, 8, lambda j, _: _loop(g * 8 + j, j), 0)
        return 0

    def _loop(row_j, j):
        row = base + row_j
        hv, tv, rv = hidx.at[j], tidx.at[j], ridx.at[j]
        c1 = pltpu.async_copy(a1.at[hv], bufa, sem)
        c2 = pltpu.async_copy(a3.at[tv], bufb, sem2)
        c3 = pltpu.async_copy(t2_sh.at[rv], bufc, sem3)
        c4 = pltpu.async_copy(t0_sh.at[rv], bufd, sem4)
        c1.wait()
        c2.wait()
        c3.wait()

        def _add(jj, _):
            for k in range(8):
                sl = pl.ds(k * 16, 16)
                bufa[jj, sl] = bufa[jj, sl] + bufb[jj, sl] + bufc[jj, sl]
            return 0

        lax.fori_loop(0, D, _add, 0)
        pltpu.sync_copy(bufa, g1_out.at[pl.ds(row * D, D)])
        c4.wait()
        pltpu.sync_copy(bufd, g0_out.at[pl.ds(row * D, D)])
        return 0

    lax.fori_loop(0, RPW // 8, _outer, 0)


def _sc_2(h2d, t2d, r2d, a1, a3, t2, t0):
    mesh = plsc.VectorSubcoreMesh(core_axis_name="c", subcore_axis_name="s")
    return pl.kernel(
        _sc2_body,
        out_type=[
            jax.ShapeDtypeStruct((MP, D), jnp.float32),
            jax.ShapeDtypeStruct((MP, D), jnp.float32),
        ],
        mesh=mesh,
        scratch_types=[
            pltpu.VMEM((8, D), jnp.int32),
            pltpu.VMEM((8, D), jnp.int32),
            pltpu.VMEM((8, D), jnp.int32),
            pltpu.VMEM((D, D), jnp.float32),
            pltpu.VMEM((D, D), jnp.float32),
            pltpu.VMEM((D, D), jnp.float32),
            pltpu.VMEM((D, D), jnp.float32),
            pltpu.VMEM_SHARED((REP2 * RPAD, D), jnp.float32),
            pltpu.VMEM_SHARED((REP2 * RPAD, D), jnp.float32),
            pltpu.SemaphoreType.DMA,
            pltpu.SemaphoreType.DMA,
            pltpu.SemaphoreType.DMA,
            pltpu.SemaphoreType.DMA,
        ],
    )(h2d, t2d, r2d, a1, a3, t2, t0)


def _tcc_body(g1, g0, g, b, wf, bf, wb, bb, pfo, pbo):
    x = _leaky(g1[...]) + g0[...]
    e1 = _ln(x, g[...], b[...])
    pfo[...] = jnp.dot(e1, wf[...], precision=_HI) + bf[...]
    pbo[...] = jnp.dot(e1, wb[...], precision=_HI) + bb[...]


def _tc_c(g1, g0, g, b, wf, bf, wb, bb):
    blk = 2048
    return pl.pallas_call(
        _tcc_body,
        grid=(MP // blk,),
        in_specs=[
            pl.BlockSpec((blk, D), lambda i: (i, 0)),
            pl.BlockSpec((blk, D), lambda i: (i, 0)),
            pl.BlockSpec((D,), lambda i: (0,)),
            pl.BlockSpec((D,), lambda i: (0,)),
            pl.BlockSpec((D, D), lambda i: (0, 0)),
            pl.BlockSpec((D,), lambda i: (0,)),
            pl.BlockSpec((D, D), lambda i: (0, 0)),
            pl.BlockSpec((D,), lambda i: (0,)),
        ],
        out_specs=[pl.BlockSpec((blk, D), lambda i: (i, 0))] * 2,
        out_shape=[jax.ShapeDtypeStruct((MP, D), jnp.float32)] * 2,
    )(g1, g0, g, b, wf, bf, wb, bb)


def _sc3_body(h2d, t2d, pf, pb, gf, gb, zeros,
              agg_out,
              hidx, tidx, buf, buf2, agg_sh, sem, sem2):
    c = lax.axis_index("c")
    s = lax.axis_index("s")
    wid = c * 16 + s

    _init_slice(zeros, agg_sh, s)
    base = wid * RPW
    plsc.subcore_barrier()

    def _outer(g, _):
        off = base + g * 8
        pltpu.sync_copy(h2d.at[pl.ds(off, 8)], hidx)
        pltpu.sync_copy(t2d.at[pl.ds(off, 8)], tidx)
        lax.fori_loop(0, 8, lambda j, _: _loop(g * 8 + j, j), 0)
        return 0

    def _loop(row_j, j):
        row = base + row_j
        hv, tv = hidx.at[j], tidx.at[j]
        c1 = pltpu.async_copy(pf.at[pl.ds(row * D, D)], buf, sem)
        c2 = pltpu.async_copy(gf.at[hv], buf2, sem2)
        c1.wait()
        pltpu.sync_copy(buf, agg_sh.at[tv], add=True)
        c3 = pltpu.async_copy(pb.at[pl.ds(row * D, D)], buf, sem)
        c2.wait()
        pltpu.sync_copy(buf2, agg_sh.at[tv], add=True)
        c4 = pltpu.async_copy(gb.at[tv], buf2, sem2)
        c3.wait()
        pltpu.sync_copy(buf, agg_sh.at[hv], add=True)
        c4.wait()
        pltpu.sync_copy(buf2, agg_sh.at[hv], add=True)
        return 0

    lax.fori_loop(0, RPW // 8, _outer, 0)

    plsc.subcore_barrier()
    _drain_slice(agg_sh, agg_out.at[c], s)


def _sc_3(h2d, t2d, pf, pb, gf, gb, zeros):
    mesh = plsc.VectorSubcoreMesh(core_axis_name="c", subcore_axis_name="s")
    return pl.kernel(
        _sc3_body,
        out_type=jax.ShapeDtypeStruct((2, N, D), jnp.float32),
        mesh=mesh,
        scratch_types=[
            pltpu.VMEM((8, D), jnp.int32),
            pltpu.VMEM((8, D), jnp.int32),
            pltpu.VMEM((D, D), jnp.float32),
            pltpu.VMEM((D, D), jnp.float32),
            pltpu.VMEM_SHARED((NP, D), jnp.float32),
            pltpu.SemaphoreType.DMA,
            pltpu.SemaphoreType.DMA,
        ],
    )(h2d, t2d, pf, pb, gf, gb, zeros)


def _tcd_body(agg, cnt, h1, g, b, h2o):
    a = agg[0] + agg[1]
    m = a / (cnt[...] + 1e-7)
    h2o[...] = _ln(_leaky(m) + h1[...], g[...], b[...])


def _tc_d(agg2, cnt, h1, g, b):
    blk = 1000
    return pl.pallas_call(
        _tcd_body,
        grid=(N // blk,),
        in_specs=[
            pl.BlockSpec((2, blk, D), lambda i: (0, i, 0)),
            pl.BlockSpec((blk, 1), lambda i: (i, 0)),
            pl.BlockSpec((blk, D), lambda i: (i, 0)),
            pl.BlockSpec((D,), lambda i: (0,)),
            pl.BlockSpec((D,), lambda i: (0,)),
        ],
        out_specs=pl.BlockSpec((blk, D), lambda i: (i, 0)),
        out_shape=jax.ShapeDtypeStruct((N, D), jnp.float32),
    )(agg2, cnt, h1, g, b)


def kernel(ht, r_tensor, entity_feat, params):
    p0 = params["layer0"]
    p1 = params["layer1"]

    pad = MP - M
    h2d = jnp.concatenate(
        [ht[:, 0], jnp.full((pad,), N, jnp.int32)]).reshape(ROWS, D)
    t2d = jnp.concatenate(
        [ht[:, 1], jnp.full((pad,), N, jnp.int32)]).reshape(ROWS, D)
    r_pad = jnp.concatenate(
        [r_tensor, jnp.full((pad,), 500, jnp.int32)])
    pos = jnp.arange(MP, dtype=jnp.int32)
    r2d = r_pad.reshape(ROWS, D)
    r2d2 = (r_pad + (pos % REP2) * RPAD).reshape(ROWS, D)
    rel512 = jnp.zeros((RPAD, D), jnp.float32).at[:500].set(params["rel_table"])
    zeros = jnp.zeros((NP, D), jnp.float32)
    ones_h = jnp.ones((D, D), jnp.float32)

    cnt2 = _sc_0(h2d, t2d, ones_h, zeros)
    cnt = _tc_r(cnt2)
    h0, gf1, gb1 = _tc_a(entity_feat, params["W_ent"], params["b_ent"],
                         p0["mpf_W"][:D], p0["mpb_W"][:D])
    tf1, tb1, t2, t0 = _tc_t(rel512, p0["mpf_W"][D:], p0["mpf_b"],
                             p0["mpb_W"][D:], p0["mpb_b"],
                             p0["eu_W"][D:2 * D], p0["eu_b"])
    agg1 = _sc_1(h2d, t2d, r2d, gf1, gb1, tf1, tb1, zeros)
    h1, a1, a3, gf2, gb2 = _tc_b(agg1, cnt, h0,
                                 p0["mp_ln_g"], p0["mp_ln_b"],
                                 p0["eu_W"][:D], p0["eu_W"][2 * D:],
                                 p1["mpf_W"][:D], p1["mpb_W"][:D])
    g1, g0 = _sc_2(h2d, t2d, r2d2, a1, a3, t2, t0)
    pf, pb = _tc_c(g1, g0, p0["eu_ln_g"], p0["eu_ln_b"],
                   p1["mpf_W"][D:], p1["mpf_b"],
                   p1["mpb_W"][D:], p1["mpb_b"])
    agg2 = _sc_3(h2d, t2d, pf, pb, gf2, gb2, zeros)
    h2 = _tc_d(agg2, cnt, h1, p1["mp_ln_g"], p1["mp_ln_b"])
    return h2
